# baseline (speedup 1.0000x reference)
import functools

import jax
import jax.numpy as jnp
from jax import lax
from jax.experimental import pallas as pl
from jax.experimental.pallas import tpu as pltpu
from jax.experimental.pallas import tpu_sc as plsc


def _make_emb_kernel(NW, NC, B, V, D):
    NBLK = (V + 127) // 128
    PB = 5
    NPASS = (NBLK + NW * PB - 1) // (NW * PB)
    DUMP = B
    mesh = plsc.VectorSubcoreMesh(core_axis_name="c", subcore_axis_name="s")

    @functools.partial(
        pl.kernel,
        mesh=mesh,
        out_type=jax.ShapeDtypeStruct((B + 32, 128), jnp.float32),
        scratch_types=[
            pltpu.VMEM((B + 16,), jnp.int32),
            pltpu.VMEM((B + 32,), jnp.int32),
            pltpu.VMEM((B + 128,), jnp.int32),
            pltpu.VMEM((PB, D, 128), jnp.float32),
            pltpu.VMEM((128, 128), jnp.float32),
            pltpu.VMEM((1, 128), jnp.int32),
            pltpu.SemaphoreType.DMA,
            pltpu.SemaphoreType.DMA,
            pltpu.SemaphoreType.DMA,
        ],
        compiler_params=pltpu.CompilerParams(
            use_tc_tiling_on_sc=True, needs_layout_passes=False,
            disable_bounds_checks=True),
    )
    def emb(x_hbm, wt_hbm, tail_hbm, out_hbm, x_v, b_all, b_pass,
            slabs, rowbuf, bidx, sem, sem2, semx):
        wid = lax.axis_index("s") * NC + lax.axis_index("c")
        blk_lo = wid * NBLK // NW
        blk_hi = (wid + 1) * NBLK // NW
        lo_v = blk_lo * 128
        hi_v = blk_hi * 128
        iota = lax.iota(jnp.int32, 16)
        np0 = jnp.minimum(blk_lo + PB, blk_hi) - blk_lo
        for bi in range(PB):
            blk0 = blk_lo + bi

            @pl.when((bi < np0) & (blk0 < NBLK - 1))
            def _pre_full():
                pltpu.async_copy(
                    wt_hbm.at[:, pl.ds(blk0 * 128, 128)], slabs.at[bi], sem)

            @pl.when((bi < np0) & (blk0 == NBLK - 1))
            def _pre_partial():
                pltpu.async_copy(tail_hbm, slabs.at[bi], sem)

        pltpu.async_copy(x_hbm.at[pl.ds(0, B // 2)],
                         x_v.at[pl.ds(0, B // 2)], semx)
        pltpu.async_copy(x_hbm.at[pl.ds(B // 2, B // 2)],
                         x_v.at[pl.ds(B // 2, B // 2)], semx)
        x_v[pl.ds(B, 16)] = jnp.zeros((16,), jnp.int32) - 1

        def scan_body(i, offv):
            vs = [x_v[pl.ds((i * 8 + k) * 16, 16)] for k in range(8)]
            ms = [(v >= lo_v) & (v < hi_v) for v in vs]
            cnts = [plsc.all_reduce_population_count(m) for m in ms]
            offs = []
            for k in range(8):
                offs.append(offv)
                offv = offv + cnts[k]
            for k in range(8):
                w16 = ((((i * 8 + k) * 16) + iota) << 17) | vs[k]
                pos = offs[k] + plsc.cumsum(ms[k].astype(jnp.int32)) - 1
                plsc.store_scatter(b_all, [pos], w16, mask=ms[k])
            return offv

        pltpu.make_async_copy(x_hbm.at[pl.ds(0, B // 2)],
                              x_v.at[pl.ds(0, B // 2)], semx).wait()
        off_half = lax.fori_loop(0, B // 256, scan_body,
                                 jnp.zeros((16,), jnp.int32))
        pltpu.make_async_copy(x_hbm.at[pl.ds(0, B // 2)],
                              x_v.at[pl.ds(0, B // 2)], semx).wait()
        m_total = lax.fori_loop(B // 256, B // 128, scan_body, off_half)[0]
        b_all[pl.ds(m_total, 16)] = jnp.zeros((16,), jnp.int32) + 0x1FFFF
        b_all[pl.ds(m_total + 16, 16)] = jnp.zeros((16,), jnp.int32) + 0x1FFFF

        for k in range(8):
            bidx[0, pl.ds(k * 16, 16)] = DUMP + ((iota + k * 16) & 31)
        pltpu.async_copy(rowbuf, out_hbm.at[bidx.at[0]], sem2)

        def pass_body(p, _):
            p_lo = blk_lo + p * PB
            p_hi = jnp.minimum(p_lo + PB, blk_hi)
            np_ = p_hi - p_lo
            lo2 = p_lo * 128
            hi2 = p_hi * 128

            for bi in range(PB):
                blk = p_lo + bi

                @pl.when((p > 0) & (bi < np_) & (blk < NBLK - 1))
                def _full():
                    pltpu.async_copy(
                        wt_hbm.at[:, pl.ds(blk * 128, 128)],
                        slabs.at[bi], sem)

                @pl.when((p > 0) & (bi < np_) & (blk == NBLK - 1))
                def _partial():
                    pltpu.async_copy(tail_hbm, slabs.at[bi], sem)

            def pscan(i, offv):
                ws = [b_all[pl.ds((i * 2 + k) * 16, 16)] for k in range(2)]
                ms = [((w & 0x1FFFF) >= lo2) & ((w & 0x1FFFF) < hi2)
                      for w in ws]
                cs = [plsc.all_reduce_population_count(m) for m in ms]
                for k in range(2):
                    pos = offv + plsc.cumsum(ms[k].astype(jnp.int32)) - 1
                    plsc.store_scatter(b_pass, [pos], ws[k], mask=ms[k])
                    offv = offv + cs[k]
                return offv

            m2 = lax.fori_loop(0, (m_total + 31) // 32, pscan,
                               jnp.zeros((16,), jnp.int32))[0]
            for k in range(8):
                b_pass[pl.ds(m2 + k * 16, 16)] = (
                    (DUMP + ((iota + k * 16) & 31)) << 17)

            for bi in range(PB):
                @pl.when(bi < np_)
                def _drain():
                    pltpu.make_async_copy(
                        wt_hbm.at[:, pl.ds(0, 128)], slabs.at[bi],
                        sem).wait()

            @pl.when(m2 > 0)
            def _pass_work():

                def group_body(g, _):
                    pltpu.make_async_copy(
                        rowbuf, out_hbm.at[bidx.at[0]], sem2).wait()
                    for k in range(8):
                        bidx[0, pl.ds(k * 16, 16)] = (
                            lax.shift_right_logical(
                                b_pass[pl.ds(g * 128 + k * 16, 16)], 17))
                    for sub in range(8):
                        gi = g * 128 + sub * 16
                        w16 = b_pass[pl.ds(gi, 16)]
                        mk = (gi + iota) < m2
                        v16 = w16 & 0x1FFFF
                        c16 = v16 & 127
                        blk16 = lax.shift_right_logical(v16, 7) - p_lo
                        slot16 = iota + sub * 16

                        dvec = iota
                        for _ in range(D):
                            val = plsc.load_gather(
                                slabs, [blk16, dvec, c16], mask=mk)
                            plsc.store_scatter(
                                rowbuf, [slot16, dvec], val, mask=mk)
                            dvec = (dvec + 1) & (D - 1)
                    pltpu.async_copy(rowbuf, out_hbm.at[bidx.at[0]],
                                     sem2)
                    return 0

                lax.fori_loop(0, (m2 + 127) // 128, group_body, 0)

            return 0

        lax.fori_loop(0, NPASS, pass_body, 0)
        pltpu.make_async_copy(rowbuf, out_hbm.at[bidx.at[0]], sem2).wait()

    return emb


def kernel(x, weight):
    B = x.shape[0]
    V, D = weight.shape
    info = plsc.get_sparse_core_info()
    NC, NS = info.num_cores, info.num_subcores
    NW = NC * NS
    wt = weight.T
    nfull = (V // 128) * 128
    tail = jnp.pad(weight[nfull:].T.astype(jnp.float32),
                   ((0, 0), (0, 128 - (V - nfull))))
    emb = _make_emb_kernel(NW, NC, B, V, D)
    out = emb(x.astype(jnp.int32), wt, tail)
    return out[:B, :D]


# speedup vs baseline: 1.0102x; 1.0102x over previous
import functools

import jax
import jax.numpy as jnp
from jax import lax
from jax.experimental import pallas as pl
from jax.experimental.pallas import tpu as pltpu
from jax.experimental.pallas import tpu_sc as plsc


def _make_emb_kernel(NW, NC, B, V, D):
    NBLK = (V + 127) // 128
    PB = 5
    NPASS = (NBLK + NW * PB - 1) // (NW * PB)
    DUMP = B
    mesh = plsc.VectorSubcoreMesh(core_axis_name="c", subcore_axis_name="s")

    @functools.partial(
        pl.kernel,
        mesh=mesh,
        out_type=jax.ShapeDtypeStruct((B + 32, 128), jnp.float32),
        scratch_types=[
            pltpu.VMEM((B + 16,), jnp.int32),
            pltpu.VMEM((B + 32,), jnp.int32),
            pltpu.VMEM((B + 128,), jnp.int32),
            pltpu.VMEM((PB, D, 128), jnp.float32),
            pltpu.VMEM((128, 128), jnp.float32),
            pltpu.VMEM((1, 128), jnp.int32),
            pltpu.SemaphoreType.DMA,
            pltpu.SemaphoreType.DMA,
            pltpu.SemaphoreType.DMA,
        ],
        compiler_params=pltpu.CompilerParams(
            use_tc_tiling_on_sc=True, needs_layout_passes=False,
            disable_bounds_checks=True),
    )
    def emb(x_hbm, wt_hbm, tail_hbm, out_hbm, x_v, b_all, b_pass,
            slabs, rowbuf, bidx, sem, sem2, semx):
        del semx
        wid = lax.axis_index("s") * NC + lax.axis_index("c")
        blk_lo = wid * NBLK // NW
        blk_hi = (wid + 1) * NBLK // NW
        lo_v = blk_lo * 128
        hi_v = blk_hi * 128
        iota = lax.iota(jnp.int32, 16)
        np0 = jnp.minimum(blk_lo + PB, blk_hi) - blk_lo
        for bi in range(PB):
            blk0 = blk_lo + bi

            @pl.when((bi < np0) & (blk0 < NBLK - 1))
            def _pre_full():
                pltpu.async_copy(
                    wt_hbm.at[:, pl.ds(blk0 * 128, 128)], slabs.at[bi], sem)

            @pl.when((bi < np0) & (blk0 == NBLK - 1))
            def _pre_partial():
                pltpu.async_copy(tail_hbm, slabs.at[bi], sem)

        pltpu.sync_copy(x_hbm, x_v.at[pl.ds(0, B)])
        x_v[pl.ds(B, 16)] = jnp.zeros((16,), jnp.int32) - 1

        def scan_body(i, offv):
            vs = [x_v[pl.ds((i * 8 + k) * 16, 16)] for k in range(8)]
            ms = [(v >= lo_v) & (v < hi_v) for v in vs]
            cnts = [plsc.all_reduce_population_count(m) for m in ms]
            offs = []
            for k in range(8):
                offs.append(offv)
                offv = offv + cnts[k]
            for k in range(8):
                w16 = ((((i * 8 + k) * 16) + iota) << 17) | vs[k]
                pos = offs[k] + plsc.cumsum(ms[k].astype(jnp.int32)) - 1
                plsc.store_scatter(b_all, [pos], w16, mask=ms[k])
            return offv

        m_total = lax.fori_loop(0, B // 128, scan_body,
                                jnp.zeros((16,), jnp.int32))[0]
        b_all[pl.ds(m_total, 16)] = jnp.zeros((16,), jnp.int32) + 0x1FFFF
        b_all[pl.ds(m_total + 16, 16)] = jnp.zeros((16,), jnp.int32) + 0x1FFFF

        for k in range(8):
            bidx[0, pl.ds(k * 16, 16)] = DUMP + ((iota + k * 16) & 31)
        pltpu.async_copy(rowbuf, out_hbm.at[bidx.at[0]], sem2)

        def pass_body(p, _):
            p_lo = blk_lo + p * PB
            p_hi = jnp.minimum(p_lo + PB, blk_hi)
            np_ = p_hi - p_lo
            lo2 = p_lo * 128
            hi2 = p_hi * 128

            for bi in range(PB):
                blk = p_lo + bi

                @pl.when((p > 0) & (bi < np_) & (blk < NBLK - 1))
                def _full():
                    pltpu.async_copy(
                        wt_hbm.at[:, pl.ds(blk * 128, 128)],
                        slabs.at[bi], sem)

                @pl.when((p > 0) & (bi < np_) & (blk == NBLK - 1))
                def _partial():
                    pltpu.async_copy(tail_hbm, slabs.at[bi], sem)

            def pscan(i, offv):
                ws = [b_all[pl.ds((i * 2 + k) * 16, 16)] for k in range(2)]
                ms = [((w & 0x1FFFF) >= lo2) & ((w & 0x1FFFF) < hi2)
                      for w in ws]
                cs = [plsc.all_reduce_population_count(m) for m in ms]
                for k in range(2):
                    pos = offv + plsc.cumsum(ms[k].astype(jnp.int32)) - 1
                    plsc.store_scatter(b_pass, [pos], ws[k], mask=ms[k])
                    offv = offv + cs[k]
                return offv

            m2 = lax.fori_loop(0, (m_total + 31) // 32, pscan,
                               jnp.zeros((16,), jnp.int32))[0]
            for k in range(8):
                b_pass[pl.ds(m2 + k * 16, 16)] = (
                    (DUMP + ((iota + k * 16) & 31)) << 17)

            for bi in range(PB):
                @pl.when(bi < np_)
                def _drain():
                    pltpu.make_async_copy(
                        wt_hbm.at[:, pl.ds(0, 128)], slabs.at[bi],
                        sem).wait()

            @pl.when(m2 > 0)
            def _pass_work():

                def group_body(g, _):
                    pltpu.make_async_copy(
                        rowbuf, out_hbm.at[bidx.at[0]], sem2).wait()
                    for k in range(8):
                        bidx[0, pl.ds(k * 16, 16)] = (
                            lax.shift_right_logical(
                                b_pass[pl.ds(g * 128 + k * 16, 16)], 17))
                    for sub in range(8):
                        gi = g * 128 + sub * 16
                        w16 = b_pass[pl.ds(gi, 16)]
                        mk = (gi + iota) < m2
                        v16 = w16 & 0x1FFFF
                        c16 = v16 & 127
                        blk16 = lax.shift_right_logical(v16, 7) - p_lo
                        slot16 = iota + sub * 16

                        dvec = iota
                        for _ in range(D):
                            val = plsc.load_gather(
                                slabs, [blk16, dvec, c16], mask=mk)
                            plsc.store_scatter(
                                rowbuf, [slot16, dvec], val, mask=mk)
                            dvec = (dvec + 1) & (D - 1)
                    pltpu.async_copy(rowbuf, out_hbm.at[bidx.at[0]],
                                     sem2)
                    return 0

                lax.fori_loop(0, (m2 + 127) // 128, group_body, 0)

            return 0

        lax.fori_loop(0, NPASS, pass_body, 0)
        pltpu.make_async_copy(rowbuf, out_hbm.at[bidx.at[0]], sem2).wait()

    return emb


def kernel(x, weight):
    B = x.shape[0]
    V, D = weight.shape
    info = plsc.get_sparse_core_info()
    NC, NS = info.num_cores, info.num_subcores
    NW = NC * NS
    wt = weight.T
    nfull = (V // 128) * 128
    tail = jnp.pad(weight[nfull:].T.astype(jnp.float32),
                   ((0, 0), (0, 128 - (V - nfull))))
    emb = _make_emb_kernel(NW, NC, B, V, D)
    out = emb(x.astype(jnp.int32), wt, tail)
    return out[:B, :D]
